# baseline (speedup 1.0000x reference)
import functools

import jax
import jax.numpy as jnp
from jax import lax
from jax.experimental import pallas as pl
from jax.experimental.pallas import tpu as pltpu
from jax.experimental.pallas import tpu_sc as plsc

B, C, H, W = 32, 768, 32, 32
HW = H * W
NC, NS = 2, 16
CH = 24
NB = 4
NCHUNK = C // CH
G = HW // 16

_MESH = plsc.VectorSubcoreMesh(core_axis_name="c", subcore_axis_name="s")


@functools.partial(
    pl.kernel,
    mesh=_MESH,
    out_type=jax.ShapeDtypeStruct((B * C, HW), jnp.float32),
    scratch_types=[
        pltpu.VMEM((HW,), jnp.int32),
        pltpu.VMEM((C // 8, 128), jnp.float32),
        pltpu.VMEM((NB, CH, HW), jnp.float32),
        pltpu.SemaphoreType.DMA((NB,)),
        pltpu.SemaphoreType.DMA((NB,)),
    ],
)
def _sc_densify(m_hbm, f_hbm, t_hbm, o_hbm, m_v, tv, bufs, sin, sout):
    wid = lax.axis_index("s") * NC + lax.axis_index("c")
    row0 = wid * C

    pltpu.sync_copy(m_hbm.at[pl.ds(wid * HW, HW)], m_v)
    pltpu.sync_copy(t_hbm, tv)

    def in_copy(n, slot):
        return pltpu.make_async_copy(
            f_hbm.at[pl.ds(row0 + n * CH, CH)], bufs.at[slot], sin.at[slot])

    def out_copy(n, slot):
        return pltpu.make_async_copy(
            bufs.at[slot], o_hbm.at[pl.ds(row0 + n * CH, CH)], sout.at[slot])

    in_copy(0, 0).start()
    in_copy(1, 1).start()

    @pl.loop(0, NCHUNK, step=NB)
    def chunk_group(base):
        for k in range(NB):
            n = base + k
            s2 = (k + 2) % NB

            @pl.when(n + 2 < NCHUNK)
            def _():
                @pl.when(n >= 2)
                def _():
                    out_copy(n - 2, s2).wait()
                in_copy(n + 2, s2).start()

            in_copy(n, k).wait()

            @pl.loop(0, CH)
            def row(r):
                c = n * CH + r
                tvec = tv[c // 8, pl.ds((c % 8) * 16, 16)]
                for j in range(G):
                    sl = pl.ds(j * 16, 16)
                    mv = m_v[sl]
                    bufs[k, r, sl] = jnp.where(mv != 0, bufs[k, r, sl], tvec)

            out_copy(n, k).start()

    for k in range(NB):
        out_copy(NCHUNK - NB + k, k).wait()


def kernel(features, active_mask, mask_token):
    f2 = features.reshape(B * C, HW)
    m1 = active_mask.astype(jnp.int32).reshape(B * HW)
    t16 = jnp.broadcast_to(
        mask_token.reshape(C // 8, 8, 1), (C // 8, 8, 16)).reshape(C // 8, 128)
    out = _sc_densify(m1, f2, t16)
    return out.reshape(B, C, H, W)


# speedup vs baseline: 1.2947x; 1.2947x over previous
import functools

import jax
import jax.numpy as jnp
from jax import lax
from jax.experimental import pallas as pl
from jax.experimental.pallas import tpu as pltpu
from jax.experimental.pallas import tpu_sc as plsc

B, C, H, W = 32, 768, 32, 32
HW = H * W
NC, NS = 2, 16
CH = 24
NB = 4
NCHUNK = C // CH
G = HW // 16

_MESH = plsc.VectorSubcoreMesh(core_axis_name="c", subcore_axis_name="s")


@functools.partial(
    pl.kernel,
    mesh=_MESH,
    out_type=jax.ShapeDtypeStruct((B * C, HW), jnp.float32),
    scratch_types=[
        pltpu.VMEM((HW,), jnp.int32),
        pltpu.VMEM((C * 16,), jnp.float32),
        pltpu.VMEM((NB, CH, HW), jnp.float32),
        pltpu.SemaphoreType.DMA((NB,)),
        pltpu.SemaphoreType.DMA((NB,)),
    ],
    compiler_params=pltpu.CompilerParams(use_tc_tiling_on_sc=True),
)
def _sc_densify(m_hbm, f_hbm, t_hbm, o_hbm, m_v, tv, bufs, sin, sout):
    wid = lax.axis_index("s") * NC + lax.axis_index("c")
    row0 = wid * C

    pltpu.sync_copy(m_hbm.at[pl.ds(wid * HW, HW)], m_v)
    pltpu.sync_copy(t_hbm, tv)

    def in_copy(n, slot):
        return pltpu.make_async_copy(
            f_hbm.at[pl.ds(row0 + n * CH, CH)], bufs.at[slot], sin.at[slot])

    def out_copy(n, slot):
        return pltpu.make_async_copy(
            bufs.at[slot], o_hbm.at[pl.ds(row0 + n * CH, CH)], sout.at[slot])

    in_copy(0, 0).start()
    in_copy(1, 1).start()

    @pl.loop(0, NCHUNK, step=NB)
    def chunk_group(base):
        for k in range(NB):
            n = base + k
            s2 = (k + 2) % NB

            @pl.when(n + 2 < NCHUNK)
            def _():
                @pl.when(n >= 2)
                def _():
                    out_copy(n - 2, s2).wait()
                in_copy(n + 2, s2).start()

            in_copy(n, k).wait()

            tvecs = [tv[pl.ds((n * CH + r) * 16, 16)] for r in range(CH)]

            @plsc.parallel_loop(0, G)
            def col_group(j):
                active = m_v[pl.ds(j * 16, 16)] != 0
                for r in range(CH):
                    sl = pl.ds(j * 16, 16)
                    bufs[k, r, sl] = jnp.where(active, bufs[k, r, sl],
                                               tvecs[r])

            out_copy(n, k).start()

    for k in range(NB):
        out_copy(NCHUNK - NB + k, k).wait()


def kernel(features, active_mask, mask_token):
    f2 = features.reshape(B * C, HW)
    m1 = active_mask.astype(jnp.int32).reshape(B * HW)
    t16 = jnp.broadcast_to(
        mask_token.reshape(C, 1), (C, 16)).reshape(C * 16)
    out = _sc_densify(m1, f2, t16)
    return out.reshape(B, C, H, W)


# speedup vs baseline: 6.0583x; 4.6795x over previous
import functools

import jax
import jax.numpy as jnp
from jax import lax
from jax.experimental import pallas as pl
from jax.experimental.pallas import tpu as pltpu
from jax.experimental.pallas import tpu_sc as plsc

B, C, H, W = 32, 768, 32, 32
HW = H * W
R = B * HW
NC, NS = 2, 16
NW = NC * NS
RW = R // NW
CH = 32
NB = 4
NCHUNK = RW // CH
G = C // 16

_MESH = plsc.VectorSubcoreMesh(core_axis_name="c", subcore_axis_name="s")


@functools.partial(
    pl.kernel,
    mesh=_MESH,
    out_type=jax.ShapeDtypeStruct((R, C), jnp.float32),
    scratch_types=[
        pltpu.VMEM((RW * 16,), jnp.int32),
        pltpu.VMEM((C,), jnp.float32),
        pltpu.VMEM((NB, CH, C), jnp.float32),
        pltpu.SemaphoreType.DMA((NB,)),
        pltpu.SemaphoreType.DMA((NB,)),
    ],
    compiler_params=pltpu.CompilerParams(use_tc_tiling_on_sc=True),
)
def _sc_densify(m_hbm, f_hbm, t_hbm, o_hbm, m_v, t_v, bufs, sin, sout):
    wid = lax.axis_index("s") * NC + lax.axis_index("c")
    row0 = wid * RW

    pltpu.sync_copy(m_hbm.at[pl.ds(row0 * 16, RW * 16)], m_v)
    pltpu.sync_copy(t_hbm, t_v)
    tvecs = [t_v[pl.ds(g * 16, 16)] for g in range(G)]

    def in_copy(n, slot):
        return pltpu.make_async_copy(
            f_hbm.at[pl.ds(row0 + n * CH, CH)], bufs.at[slot], sin.at[slot])

    def out_copy(n, slot):
        return pltpu.make_async_copy(
            bufs.at[slot], o_hbm.at[pl.ds(row0 + n * CH, CH)], sout.at[slot])

    in_copy(0, 0).start()
    in_copy(1, 1).start()

    @pl.loop(0, NCHUNK, step=NB)
    def chunk_group(base):
        for k in range(NB):
            n = base + k
            s2 = (k + 2) % NB

            @pl.when(n + 2 < NCHUNK)
            def _():
                @pl.when(n >= 2)
                def _():
                    out_copy(n - 2, s2).wait()
                in_copy(n + 2, s2).start()

            in_copy(n, k).wait()

            @plsc.parallel_loop(0, CH)
            def row(r):
                active = m_v[pl.ds((n * CH + r) * 16, 16)] != 0
                for g in range(G):
                    sl = pl.ds(g * 16, 16)
                    bufs[k, r, sl] = jnp.where(active, bufs[k, r, sl],
                                               tvecs[g])

            out_copy(n, k).start()

    for k in range(NB):
        out_copy(NCHUNK - NB + k, k).wait()


def kernel(features, active_mask, mask_token):
    fT = jnp.transpose(features, (0, 2, 3, 1)).reshape(R, C)
    m16 = jnp.broadcast_to(
        active_mask.reshape(R, 1).astype(jnp.int32), (R, 16)).reshape(R * 16)
    t1 = mask_token.reshape(C)
    outT = _sc_densify(m16, fT, t1)
    return jnp.transpose(outT.reshape(B, H, W, C), (0, 3, 1, 2))


# speedup vs baseline: 7.6116x; 1.2564x over previous
import functools

import jax
import jax.numpy as jnp
from jax import lax
from jax.experimental import pallas as pl
from jax.experimental.pallas import tpu as pltpu
from jax.experimental.pallas import tpu_sc as plsc

B, C, H, W = 32, 768, 32, 32
HW = H * W
R = B * HW
NC, NS = 2, 16
NW = NC * NS
RW = R // NW
CH = 32
NB = 4
NCHUNK = RW // CH
G = C // 16

_MESH = plsc.VectorSubcoreMesh(core_axis_name="c", subcore_axis_name="s")


@functools.partial(
    pl.kernel,
    mesh=_MESH,
    out_type=jax.ShapeDtypeStruct((R, C), jnp.float32),
    scratch_types=[
        pltpu.VMEM((RW // 8, 128), jnp.int32),
        pltpu.VMEM((C,), jnp.float32),
        pltpu.VMEM((NB, CH, C), jnp.float32),
        pltpu.SemaphoreType.DMA((NB,)),
        pltpu.SemaphoreType.DMA((NB,)),
    ],
    compiler_params=pltpu.CompilerParams(use_tc_tiling_on_sc=True),
)
def _sc_densify(m_hbm, f_hbm, t_hbm, o_hbm, m_v, t_v, bufs, sin, sout):
    wid = lax.axis_index("s") * NC + lax.axis_index("c")
    row0 = wid * RW

    pltpu.sync_copy(m_hbm.at[pl.ds(wid * (RW // 8), RW // 8)], m_v)
    pltpu.sync_copy(t_hbm, t_v)
    tvecs = [t_v[pl.ds(g * 16, 16)] for g in range(G)]

    def in_copy(n, slot):
        return pltpu.make_async_copy(
            f_hbm.at[pl.ds(row0 + n * CH, CH)], bufs.at[slot], sin.at[slot])

    def out_copy(n, slot):
        return pltpu.make_async_copy(
            bufs.at[slot], o_hbm.at[pl.ds(row0 + n * CH, CH)], sout.at[slot])

    in_copy(0, 0).start()
    in_copy(1, 1).start()

    @pl.loop(0, NCHUNK, step=NB)
    def chunk_group(base):
        for k in range(NB):
            n = base + k
            s2 = (k + 2) % NB

            @pl.when(n + 2 < NCHUNK)
            def _():
                @pl.when(n >= 2)
                def _():
                    out_copy(n - 2, s2).wait()
                in_copy(n + 2, s2).start()

            in_copy(n, k).wait()

            @plsc.parallel_loop(0, CH)
            def row(r):
                rr = n * CH + r
                active = m_v[rr // 8, pl.ds((rr % 8) * 16, 16)] != 0
                for g in range(G):
                    sl = pl.ds(g * 16, 16)
                    bufs[k, r, sl] = jnp.where(active, bufs[k, r, sl],
                                               tvecs[g])

            out_copy(n, k).start()

    for k in range(NB):
        out_copy(NCHUNK - NB + k, k).wait()


def kernel(features, active_mask, mask_token):
    fT = jnp.transpose(features, (0, 2, 3, 1)).reshape(R, C)
    m16 = jnp.broadcast_to(
        active_mask.reshape(R // 8, 8, 1).astype(jnp.int32),
        (R // 8, 8, 16)).reshape(R // 8, 128)
    t1 = mask_token.reshape(C)
    outT = _sc_densify(m16, fT, t1)
    return jnp.transpose(outT.reshape(B, H, W, C), (0, 3, 1, 2))


# speedup vs baseline: 7.6606x; 1.0064x over previous
import functools

import jax
import jax.numpy as jnp
from jax import lax
from jax.experimental import pallas as pl
from jax.experimental.pallas import tpu as pltpu
from jax.experimental.pallas import tpu_sc as plsc

B, C, H, W = 32, 768, 32, 32
HW = H * W
R = B * HW
NC, NS = 2, 16
NW = NC * NS
RW = R // NW
CH = 16
NB = 8
LA = NB - 2
NCHUNK = RW // CH
G = C // 16

_MESH = plsc.VectorSubcoreMesh(core_axis_name="c", subcore_axis_name="s")


@functools.partial(
    pl.kernel,
    mesh=_MESH,
    out_type=jax.ShapeDtypeStruct((R, C), jnp.float32),
    scratch_types=[
        pltpu.VMEM((RW // 8, 128), jnp.int32),
        pltpu.VMEM((C,), jnp.float32),
        pltpu.VMEM((NB, CH, C), jnp.float32),
        pltpu.SemaphoreType.DMA((NB,)),
        pltpu.SemaphoreType.DMA((NB,)),
    ],
    compiler_params=pltpu.CompilerParams(use_tc_tiling_on_sc=True),
)
def _sc_densify(m_hbm, f_hbm, t_hbm, o_hbm, m_v, t_v, bufs, sin, sout):
    wid = lax.axis_index("s") * NC + lax.axis_index("c")
    row0 = wid * RW

    pltpu.sync_copy(m_hbm.at[pl.ds(wid * (RW // 8), RW // 8)], m_v)
    pltpu.sync_copy(t_hbm, t_v)
    tvecs = [t_v[pl.ds(g * 16, 16)] for g in range(G)]

    def in_copy(n, slot):
        return pltpu.make_async_copy(
            f_hbm.at[pl.ds(row0 + n * CH, CH)], bufs.at[slot], sin.at[slot])

    def out_copy(n, slot):
        return pltpu.make_async_copy(
            bufs.at[slot], o_hbm.at[pl.ds(row0 + n * CH, CH)], sout.at[slot])

    for k in range(LA):
        in_copy(k, k).start()

    @pl.loop(0, NCHUNK, step=NB)
    def chunk_group(base):
        for k in range(NB):
            n = base + k
            s2 = (k + LA) % NB

            @pl.when(n + LA < NCHUNK)
            def _():
                @pl.when(n + LA >= NB)
                def _():
                    out_copy(n + LA - NB, s2).wait()
                in_copy(n + LA, s2).start()

            in_copy(n, k).wait()

            @plsc.parallel_loop(0, CH)
            def row(r):
                rr = n * CH + r
                active = m_v[rr // 8, pl.ds((rr % 8) * 16, 16)] != 0
                for g in range(G):
                    sl = pl.ds(g * 16, 16)
                    bufs[k, r, sl] = jnp.where(active, bufs[k, r, sl],
                                               tvecs[g])

            out_copy(n, k).start()

    for k in range(NB):
        out_copy(NCHUNK - NB + k, k).wait()


def kernel(features, active_mask, mask_token):
    fT = jnp.transpose(features, (0, 2, 3, 1)).reshape(R, C)
    m16 = jnp.broadcast_to(
        active_mask.reshape(R // 8, 8, 1).astype(jnp.int32),
        (R // 8, 8, 16)).reshape(R // 8, 128)
    t1 = mask_token.reshape(C)
    outT = _sc_densify(m16, fT, t1)
    return jnp.transpose(outT.reshape(B, H, W, C), (0, 3, 1, 2))
